# Initial kernel scaffold; baseline (speedup 1.0000x reference)
import jax
import jax.numpy as jnp
from jax.experimental import pallas as pl


def kernel(x, edge_index, W1, b1, g1, bt1, W2, b2, g2, bt2, Wfc, bfc):
    raise NotImplementedError("write your pallas kernel here")


# speedup vs baseline: 13.6530x; 13.6530x over previous
import functools

import jax
import jax.numpy as jnp
from jax import lax
from jax.experimental import pallas as pl
from jax.experimental.pallas import tpu as pltpu
from jax.experimental.pallas import tpu_sc as plsc

N = 10000
E = 320000
EPAD = 327680
ROWS128 = EPAD // 128
CH_SC = ROWS128 // 16
CH_DEG = ROWS128 // 32
ACC = 10112
HALF = 128


def _mesh():
    return plsc.VectorSubcoreMesh(core_axis_name="c", subcore_axis_name="s")


def _deg_parts(dst2d, ones128, zer128):
    @functools.partial(
        pl.kernel,
        out_type=jax.ShapeDtypeStruct((2 * N, HALF), jnp.float32),
        mesh=_mesh(),
        scratch_types=[
            pltpu.VMEM((CH_DEG, 128), jnp.int32),
            pltpu.VMEM((128, HALF), jnp.float32),
            pltpu.VMEM_SHARED((ACC, HALF), jnp.float32),
        ],
    )
    def kern(dst_hbm, ones_hbm, zer_hbm, out_hbm, idx_v, ones_v, acc):
        c = lax.axis_index("c")
        s = lax.axis_index("s")
        wid = s * 2 + c
        pltpu.sync_copy(zer_hbm, acc.at[pl.ds(s * 632, 632)])
        pltpu.sync_copy(ones_hbm, ones_v)
        pltpu.sync_copy(dst_hbm.at[pl.ds(wid * CH_DEG, CH_DEG)], idx_v)
        plsc.subcore_barrier()

        @pl.loop(0, CH_DEG)
        def _(j):
            pltpu.sync_copy(ones_v, acc.at[idx_v.at[j]], add=True)

        plsc.subcore_barrier()

        @pl.when(s < 10)
        def _():
            pltpu.sync_copy(acc.at[pl.ds(s * 1000, 1000)],
                            out_hbm.at[pl.ds(c * N + s * 1000, 1000)])

    return kern(dst2d, ones128, zer128)


def _sc_scatter(zA, zB, src2d, dst2d, zer128):
    out_t = jax.ShapeDtypeStruct((N, HALF), jnp.float32)

    @functools.partial(
        pl.kernel,
        out_type=[out_t, out_t],
        mesh=_mesh(),
        scratch_types=[
            pltpu.VMEM((16, 128), jnp.int32),
            pltpu.VMEM((16, 128), jnp.int32),
            pltpu.VMEM((128, HALF), jnp.float32),
            pltpu.VMEM_SHARED((ACC, HALF), jnp.float32),
            pltpu.SemaphoreType.DMA,
        ],
    )
    def kern(zA_hbm, zB_hbm, src_hbm, dst_hbm, zer_hbm, outA, outB,
             src_v, dst_v, rows_v, acc, sem):
        c = lax.axis_index("c")
        s = lax.axis_index("s")
        pltpu.sync_copy(zer_hbm, acc.at[pl.ds(s * 632, 632)])
        plsc.subcore_barrier()

        @pl.loop(0, CH_SC // 16)
        def _(b):
            pltpu.sync_copy(src_hbm.at[pl.ds(s * CH_SC + b * 16, 16)], src_v)
            pltpu.sync_copy(dst_hbm.at[pl.ds(s * CH_SC + b * 16, 16)], dst_v)

            @pl.loop(0, 16)
            def _(j):
                @pl.when(c == 0)
                def _():
                    pltpu.async_copy(zA_hbm.at[src_v.at[j]], rows_v,
                                     sem).wait()

                @pl.when(c == 1)
                def _():
                    pltpu.async_copy(zB_hbm.at[src_v.at[j]], rows_v,
                                     sem).wait()

                pltpu.sync_copy(rows_v, acc.at[dst_v.at[j]], add=True)

        plsc.subcore_barrier()

        @pl.when((c == 0) & (s < 10))
        def _():
            pltpu.sync_copy(acc.at[pl.ds(s * 1000, 1000)],
                            outA.at[pl.ds(s * 1000, 1000)])

        @pl.when((c == 1) & (s < 10))
        def _():
            pltpu.sync_copy(acc.at[pl.ds(s * 1000, 1000)],
                            outB.at[pl.ds(s * 1000, 1000)])

    return kern(zA, zB, src2d, dst2d, zer128)


def _dinv(deg_ref):
    d = deg_ref[0:N, 0:1] + deg_ref[N:2 * N, 0:1] + 1.0
    return lax.rsqrt(d)


def _mm_scale1(x, W1, degflat):
    def body(x_ref, w_ref, deg_ref, zA_ref, zB_ref):
        h = jnp.dot(x_ref[...], w_ref[...], preferred_element_type=jnp.float32)
        z = h * _dinv(deg_ref)
        zA_ref[...] = z[:, :HALF]
        zB_ref[...] = z[:, HALF:]

    out_t = jax.ShapeDtypeStruct((N, HALF), jnp.float32)
    return pl.pallas_call(body, out_shape=[out_t, out_t])(x, W1, degflat)


def _bn_half(S, z, degflat, b, g, bt):
    def body(S_ref, z_ref, deg_ref, b_ref, g_ref, bt_ref, a_ref):
        p = (S_ref[...] + z_ref[...]) * _dinv(deg_ref) + b_ref[...]
        mu = jnp.mean(p, axis=0, keepdims=True)
        q = p - mu
        var = jnp.mean(q * q, axis=0, keepdims=True)
        a_ref[...] = jnp.maximum(
            q * lax.rsqrt(var + 1e-5) * g_ref[...] + bt_ref[...], 0.0)

    out_t = jax.ShapeDtypeStruct((N, HALF), jnp.float32)
    return pl.pallas_call(body, out_shape=out_t)(S, z, degflat, b, g, bt)


def _mm_scale2(aA, aB, W2, degflat):
    def body(aA_ref, aB_ref, w_ref, deg_ref, zA_ref, zB_ref):
        a = jnp.concatenate([aA_ref[...], aB_ref[...]], axis=1)
        h = jnp.dot(a, w_ref[...], preferred_element_type=jnp.float32)
        z = h * _dinv(deg_ref)
        zA_ref[...] = z[:, :HALF]
        zB_ref[...] = z[:, HALF:]

    out_t = jax.ShapeDtypeStruct((N, HALF), jnp.float32)
    return pl.pallas_call(body, out_shape=[out_t, out_t])(aA, aB, W2, degflat)


def _final(aA, aB, Wfc, bfc):
    def body(aA_ref, aB_ref, w_ref, b_ref, y_ref):
        a = jnp.concatenate([aA_ref[...], aB_ref[...]], axis=1)
        y_ref[...] = jnp.dot(a, w_ref[...],
                             preferred_element_type=jnp.float32) + b_ref[...]

    out_t = jax.ShapeDtypeStruct((N, Wfc.shape[1]), jnp.float32)
    return pl.pallas_call(body, out_shape=out_t)(aA, aB, Wfc, bfc)


def kernel(x, edge_index, W1, b1, g1, bt1, W2, b2, g2, bt2, Wfc, bfc):
    ei = edge_index.astype(jnp.int32)
    pad = EPAD - E
    pad_src = (jnp.arange(pad, dtype=jnp.int32) * 97) % N
    pad_dst = N + (jnp.arange(pad, dtype=jnp.int32) % 48)
    src2d = jnp.concatenate([ei[0], pad_src]).reshape(ROWS128, 128)
    dst2d = jnp.concatenate([ei[1], pad_dst]).reshape(ROWS128, 128)

    ones128 = jnp.ones((128, HALF), jnp.float32)
    zer128 = jnp.zeros((632, HALF), jnp.float32)

    degflat = _deg_parts(dst2d, ones128, zer128)

    z1A, z1B = _mm_scale1(x, W1, degflat)
    S1A, S1B = _sc_scatter(z1A, z1B, src2d, dst2d, zer128)
    a1A = _bn_half(S1A, z1A, degflat, b1[:HALF].reshape(1, -1),
                   g1[:HALF].reshape(1, -1), bt1[:HALF].reshape(1, -1))
    a1B = _bn_half(S1B, z1B, degflat, b1[HALF:].reshape(1, -1),
                   g1[HALF:].reshape(1, -1), bt1[HALF:].reshape(1, -1))

    z2A, z2B = _mm_scale2(a1A, a1B, W2, degflat)
    S2A, S2B = _sc_scatter(z2A, z2B, src2d, dst2d, zer128)
    a2A = _bn_half(S2A, z2A, degflat, b2[:HALF].reshape(1, -1),
                   g2[:HALF].reshape(1, -1), bt2[:HALF].reshape(1, -1))
    a2B = _bn_half(S2B, z2B, degflat, b2[HALF:].reshape(1, -1),
                   g2[HALF:].reshape(1, -1), bt2[HALF:].reshape(1, -1))

    return _final(a2A, a2B, Wfc, bfc.reshape(1, -1))


# speedup vs baseline: 15.3640x; 1.1253x over previous
import functools

import jax
import jax.numpy as jnp
from jax import lax
from jax.experimental import pallas as pl
from jax.experimental.pallas import tpu as pltpu
from jax.experimental.pallas import tpu_sc as plsc

N = 10000
E = 320000
EPAD = 327680
ROWS128 = EPAD // 128
CH_SC = ROWS128 // 16
CH_DEG = ROWS128 // 32
ACC = 10112
HALF = 128


def _mesh():
    return plsc.VectorSubcoreMesh(core_axis_name="c", subcore_axis_name="s")


def _deg_parts(dst2d, ones128, zer128):
    @functools.partial(
        pl.kernel,
        out_type=jax.ShapeDtypeStruct((2 * N, HALF), jnp.float32),
        mesh=_mesh(),
        scratch_types=[
            pltpu.VMEM((CH_DEG, 128), jnp.int32),
            pltpu.VMEM((128, HALF), jnp.float32),
            pltpu.VMEM_SHARED((ACC, HALF), jnp.float32),
        ],
    )
    def kern(dst_hbm, ones_hbm, zer_hbm, out_hbm, idx_v, ones_v, acc):
        c = lax.axis_index("c")
        s = lax.axis_index("s")
        wid = s * 2 + c
        pltpu.sync_copy(zer_hbm, acc.at[pl.ds(s * 632, 632)])
        pltpu.sync_copy(ones_hbm, ones_v)
        pltpu.sync_copy(dst_hbm.at[pl.ds(wid * CH_DEG, CH_DEG)], idx_v)
        plsc.subcore_barrier()

        @pl.loop(0, CH_DEG)
        def _(j):
            pltpu.sync_copy(ones_v, acc.at[idx_v.at[j]], add=True)

        plsc.subcore_barrier()

        @pl.when(s < 10)
        def _():
            pltpu.sync_copy(acc.at[pl.ds(s * 1000, 1000)],
                            out_hbm.at[pl.ds(c * N + s * 1000, 1000)])

    return kern(dst2d, ones128, zer128)


def _sc_scatter(zA, zB, src2d, dst2d, zer128):
    out_t = jax.ShapeDtypeStruct((N, HALF), jnp.float32)

    @functools.partial(
        pl.kernel,
        out_type=[out_t, out_t],
        mesh=_mesh(),
        scratch_types=[
            pltpu.VMEM((16, 128), jnp.int32),
            pltpu.VMEM((16, 128), jnp.int32),
            pltpu.VMEM((128, HALF), jnp.float32),
            pltpu.VMEM((128, HALF), jnp.float32),
            pltpu.VMEM_SHARED((ACC, HALF), jnp.float32),
            pltpu.SemaphoreType.DMA,
            pltpu.SemaphoreType.DMA,
        ],
    )
    def kern(zA_hbm, zB_hbm, src_hbm, dst_hbm, zer_hbm, outA, outB,
             src_v, dst_v, rows0, rows1, acc, sem0, sem1):
        c = lax.axis_index("c")
        s = lax.axis_index("s")
        pltpu.sync_copy(zer_hbm, acc.at[pl.ds(s * 632, 632)])
        plsc.subcore_barrier()

        @pl.loop(0, CH_SC // 2)
        def _(p):
            @pl.when(p % 8 == 0)
            def _():
                blk = p // 8
                pltpu.sync_copy(
                    src_hbm.at[pl.ds(s * CH_SC + blk * 16, 16)], src_v)
                pltpu.sync_copy(
                    dst_hbm.at[pl.ds(s * CH_SC + blk * 16, 16)], dst_v)

            r0 = (p % 8) * 2
            r1 = r0 + 1

            @pl.when(c == 0)
            def _():
                cp0 = pltpu.async_copy(zA_hbm.at[src_v.at[r0]], rows0, sem0)
                cp1 = pltpu.async_copy(zA_hbm.at[src_v.at[r1]], rows1, sem1)
                cp0.wait()
                pltpu.sync_copy(rows0, acc.at[dst_v.at[r0]], add=True)
                cp1.wait()
                pltpu.sync_copy(rows1, acc.at[dst_v.at[r1]], add=True)

            @pl.when(c == 1)
            def _():
                cp0 = pltpu.async_copy(zB_hbm.at[src_v.at[r0]], rows0, sem0)
                cp1 = pltpu.async_copy(zB_hbm.at[src_v.at[r1]], rows1, sem1)
                cp0.wait()
                pltpu.sync_copy(rows0, acc.at[dst_v.at[r0]], add=True)
                cp1.wait()
                pltpu.sync_copy(rows1, acc.at[dst_v.at[r1]], add=True)

        plsc.subcore_barrier()

        @pl.when((c == 0) & (s < 10))
        def _():
            pltpu.sync_copy(acc.at[pl.ds(s * 1000, 1000)],
                            outA.at[pl.ds(s * 1000, 1000)])

        @pl.when((c == 1) & (s < 10))
        def _():
            pltpu.sync_copy(acc.at[pl.ds(s * 1000, 1000)],
                            outB.at[pl.ds(s * 1000, 1000)])

    return kern(zA, zB, src2d, dst2d, zer128)


def _dinv(deg_ref):
    d = deg_ref[0:N, 0:1] + deg_ref[N:2 * N, 0:1] + 1.0
    return lax.rsqrt(d)


def _mm_scale1(x, W1, degflat):
    def body(x_ref, w_ref, deg_ref, zA_ref, zB_ref):
        h = jnp.dot(x_ref[...], w_ref[...], preferred_element_type=jnp.float32)
        z = h * _dinv(deg_ref)
        zA_ref[...] = z[:, :HALF]
        zB_ref[...] = z[:, HALF:]

    out_t = jax.ShapeDtypeStruct((N, HALF), jnp.float32)
    return pl.pallas_call(body, out_shape=[out_t, out_t])(x, W1, degflat)


def _bn_half(S, z, degflat, b, g, bt):
    def body(S_ref, z_ref, deg_ref, b_ref, g_ref, bt_ref, a_ref):
        p = (S_ref[...] + z_ref[...]) * _dinv(deg_ref) + b_ref[...]
        mu = jnp.mean(p, axis=0, keepdims=True)
        q = p - mu
        var = jnp.mean(q * q, axis=0, keepdims=True)
        a_ref[...] = jnp.maximum(
            q * lax.rsqrt(var + 1e-5) * g_ref[...] + bt_ref[...], 0.0)

    out_t = jax.ShapeDtypeStruct((N, HALF), jnp.float32)
    return pl.pallas_call(body, out_shape=out_t)(S, z, degflat, b, g, bt)


def _mm_scale2(aA, aB, W2, degflat):
    def body(aA_ref, aB_ref, w_ref, deg_ref, zA_ref, zB_ref):
        a = jnp.concatenate([aA_ref[...], aB_ref[...]], axis=1)
        h = jnp.dot(a, w_ref[...], preferred_element_type=jnp.float32)
        z = h * _dinv(deg_ref)
        zA_ref[...] = z[:, :HALF]
        zB_ref[...] = z[:, HALF:]

    out_t = jax.ShapeDtypeStruct((N, HALF), jnp.float32)
    return pl.pallas_call(body, out_shape=[out_t, out_t])(aA, aB, W2, degflat)


def _final(aA, aB, Wfc, bfc):
    def body(aA_ref, aB_ref, w_ref, b_ref, y_ref):
        a = jnp.concatenate([aA_ref[...], aB_ref[...]], axis=1)
        y_ref[...] = jnp.dot(a, w_ref[...],
                             preferred_element_type=jnp.float32) + b_ref[...]

    out_t = jax.ShapeDtypeStruct((N, Wfc.shape[1]), jnp.float32)
    return pl.pallas_call(body, out_shape=out_t)(aA, aB, Wfc, bfc)


def kernel(x, edge_index, W1, b1, g1, bt1, W2, b2, g2, bt2, Wfc, bfc):
    ei = edge_index.astype(jnp.int32)
    pad = EPAD - E
    pad_src = (jnp.arange(pad, dtype=jnp.int32) * 97) % N
    pad_dst = N + (jnp.arange(pad, dtype=jnp.int32) % 48)
    src2d = jnp.concatenate([ei[0], pad_src]).reshape(ROWS128, 128)
    dst2d = jnp.concatenate([ei[1], pad_dst]).reshape(ROWS128, 128)

    ones128 = jnp.ones((128, HALF), jnp.float32)
    zer128 = jnp.zeros((632, HALF), jnp.float32)

    degflat = _deg_parts(dst2d, ones128, zer128)

    z1A, z1B = _mm_scale1(x, W1, degflat)
    S1A, S1B = _sc_scatter(z1A, z1B, src2d, dst2d, zer128)
    a1A = _bn_half(S1A, z1A, degflat, b1[:HALF].reshape(1, -1),
                   g1[:HALF].reshape(1, -1), bt1[:HALF].reshape(1, -1))
    a1B = _bn_half(S1B, z1B, degflat, b1[HALF:].reshape(1, -1),
                   g1[HALF:].reshape(1, -1), bt1[HALF:].reshape(1, -1))

    z2A, z2B = _mm_scale2(a1A, a1B, W2, degflat)
    S2A, S2B = _sc_scatter(z2A, z2B, src2d, dst2d, zer128)
    a2A = _bn_half(S2A, z2A, degflat, b2[:HALF].reshape(1, -1),
                   g2[:HALF].reshape(1, -1), bt2[:HALF].reshape(1, -1))
    a2B = _bn_half(S2B, z2B, degflat, b2[HALF:].reshape(1, -1),
                   g2[HALF:].reshape(1, -1), bt2[HALF:].reshape(1, -1))

    return _final(a2A, a2B, Wfc, bfc.reshape(1, -1))


# speedup vs baseline: 15.5131x; 1.0097x over previous
import functools

import jax
import jax.numpy as jnp
from jax import lax
from jax.experimental import pallas as pl
from jax.experimental.pallas import tpu as pltpu
from jax.experimental.pallas import tpu_sc as plsc

N = 10000
E = 320000
EPAD = 327680
ROWS128 = EPAD // 128
CH_SC = ROWS128 // 16
CH_DEG = ROWS128 // 32
ACC = 10112
HALF = 128


def _mesh():
    return plsc.VectorSubcoreMesh(core_axis_name="c", subcore_axis_name="s")


def _deg_parts(dst2d, ones128, zer128):
    @functools.partial(
        pl.kernel,
        out_type=jax.ShapeDtypeStruct((2 * N, HALF), jnp.float32),
        mesh=_mesh(),
        scratch_types=[
            pltpu.VMEM((CH_DEG, 128), jnp.int32),
            pltpu.VMEM((128, HALF), jnp.float32),
            pltpu.VMEM_SHARED((ACC, HALF), jnp.float32),
        ],
    )
    def kern(dst_hbm, ones_hbm, zer_hbm, out_hbm, idx_v, ones_v, acc):
        c = lax.axis_index("c")
        s = lax.axis_index("s")
        wid = s * 2 + c
        pltpu.sync_copy(zer_hbm, acc.at[pl.ds(s * 632, 632)])
        pltpu.sync_copy(ones_hbm, ones_v)
        pltpu.sync_copy(dst_hbm.at[pl.ds(wid * CH_DEG, CH_DEG)], idx_v)
        plsc.subcore_barrier()

        @pl.loop(0, CH_DEG)
        def _(j):
            pltpu.sync_copy(ones_v, acc.at[idx_v.at[j]], add=True)

        plsc.subcore_barrier()

        @pl.when(s < 10)
        def _():
            pltpu.sync_copy(acc.at[pl.ds(s * 1000, 1000)],
                            out_hbm.at[pl.ds(c * N + s * 1000, 1000)])

    return kern(dst2d, ones128, zer128)


def _sc_scatter(zA, zB, src2d, dst2d, zer128):
    out_t = jax.ShapeDtypeStruct((N, HALF), jnp.float32)

    @functools.partial(
        pl.kernel,
        out_type=[out_t, out_t],
        mesh=_mesh(),
        scratch_types=[
            pltpu.VMEM((16, 128), jnp.int32),
            pltpu.VMEM((16, 128), jnp.int32),
            pltpu.VMEM((128, HALF), jnp.float32),
            pltpu.VMEM((128, HALF), jnp.float32),
            pltpu.VMEM_SHARED((ACC, HALF), jnp.float32),
            pltpu.SemaphoreType.DMA,
            pltpu.SemaphoreType.DMA,
            pltpu.SemaphoreType.DMA,
            pltpu.SemaphoreType.DMA,
        ],
    )
    def kern(zA_hbm, zB_hbm, src_hbm, dst_hbm, zer_hbm, outA, outB,
             src_v, dst_v, rows0, rows1, acc, sem0, sem1, sem2, sem3):
        c = lax.axis_index("c")
        s = lax.axis_index("s")
        pltpu.sync_copy(zer_hbm, acc.at[pl.ds(s * 632, 632)])
        plsc.subcore_barrier()

        @pl.loop(0, CH_SC // 2)
        def _(p):
            @pl.when(p > 0)
            def _():
                pltpu.make_async_copy(
                    zA_hbm.at[pl.ds(0, 128)], rows0, sem2).wait()
                pltpu.make_async_copy(
                    zA_hbm.at[pl.ds(0, 128)], rows1, sem3).wait()

            @pl.when(p % 8 == 0)
            def _():
                blk = p // 8
                pltpu.sync_copy(
                    src_hbm.at[pl.ds(s * CH_SC + blk * 16, 16)], src_v)
                pltpu.sync_copy(
                    dst_hbm.at[pl.ds(s * CH_SC + blk * 16, 16)], dst_v)

            r0 = (p % 8) * 2
            r1 = r0 + 1

            @pl.when(c == 0)
            def _():
                cp0 = pltpu.async_copy(zA_hbm.at[src_v.at[r0]], rows0, sem0)
                cp1 = pltpu.async_copy(zA_hbm.at[src_v.at[r1]], rows1, sem1)
                cp0.wait()
                pltpu.async_copy(rows0, acc.at[dst_v.at[r0]], sem2, add=True)
                cp1.wait()
                pltpu.async_copy(rows1, acc.at[dst_v.at[r1]], sem3, add=True)

            @pl.when(c == 1)
            def _():
                cp0 = pltpu.async_copy(zB_hbm.at[src_v.at[r0]], rows0, sem0)
                cp1 = pltpu.async_copy(zB_hbm.at[src_v.at[r1]], rows1, sem1)
                cp0.wait()
                pltpu.async_copy(rows0, acc.at[dst_v.at[r0]], sem2, add=True)
                cp1.wait()
                pltpu.async_copy(rows1, acc.at[dst_v.at[r1]], sem3, add=True)

        pltpu.make_async_copy(zA_hbm.at[pl.ds(0, 128)], rows0, sem2).wait()
        pltpu.make_async_copy(zA_hbm.at[pl.ds(0, 128)], rows1, sem3).wait()
        plsc.subcore_barrier()

        @pl.when((c == 0) & (s < 10))
        def _():
            pltpu.sync_copy(acc.at[pl.ds(s * 1000, 1000)],
                            outA.at[pl.ds(s * 1000, 1000)])

        @pl.when((c == 1) & (s < 10))
        def _():
            pltpu.sync_copy(acc.at[pl.ds(s * 1000, 1000)],
                            outB.at[pl.ds(s * 1000, 1000)])

    return kern(zA, zB, src2d, dst2d, zer128)


def _dinv(deg_ref):
    d = deg_ref[0:N, 0:1] + deg_ref[N:2 * N, 0:1] + 1.0
    return lax.rsqrt(d)


def _mm1(x, W1):
    def body(x_ref, w_ref, hA_ref, hB_ref):
        h = jnp.dot(x_ref[...], w_ref[...], preferred_element_type=jnp.float32)
        hA_ref[...] = h[:, :HALF]
        hB_ref[...] = h[:, HALF:]

    out_t = jax.ShapeDtypeStruct((N, HALF), jnp.float32)
    return pl.pallas_call(body, out_shape=[out_t, out_t])(x, W1)


def _scale(hA, hB, degflat):
    def body(hA_ref, hB_ref, deg_ref, zA_ref, zB_ref):
        dinv = _dinv(deg_ref)
        zA_ref[...] = hA_ref[...] * dinv
        zB_ref[...] = hB_ref[...] * dinv

    out_t = jax.ShapeDtypeStruct((N, HALF), jnp.float32)
    return pl.pallas_call(body, out_shape=[out_t, out_t])(hA, hB, degflat)


def _bn_half(S, z, degflat, b, g, bt):
    def body(S_ref, z_ref, deg_ref, b_ref, g_ref, bt_ref, a_ref):
        p = (S_ref[...] + z_ref[...]) * _dinv(deg_ref) + b_ref[...]
        mu = jnp.mean(p, axis=0, keepdims=True)
        q = p - mu
        var = jnp.mean(q * q, axis=0, keepdims=True)
        a_ref[...] = jnp.maximum(
            q * lax.rsqrt(var + 1e-5) * g_ref[...] + bt_ref[...], 0.0)

    out_t = jax.ShapeDtypeStruct((N, HALF), jnp.float32)
    return pl.pallas_call(body, out_shape=out_t)(S, z, degflat, b, g, bt)


def _mm_scale2(aA, aB, W2, degflat):
    def body(aA_ref, aB_ref, w_ref, deg_ref, zA_ref, zB_ref):
        a = jnp.concatenate([aA_ref[...], aB_ref[...]], axis=1)
        h = jnp.dot(a, w_ref[...], preferred_element_type=jnp.float32)
        z = h * _dinv(deg_ref)
        zA_ref[...] = z[:, :HALF]
        zB_ref[...] = z[:, HALF:]

    out_t = jax.ShapeDtypeStruct((N, HALF), jnp.float32)
    return pl.pallas_call(body, out_shape=[out_t, out_t])(aA, aB, W2, degflat)


def _final(aA, aB, Wfc, bfc):
    def body(aA_ref, aB_ref, w_ref, b_ref, y_ref):
        a = jnp.concatenate([aA_ref[...], aB_ref[...]], axis=1)
        y_ref[...] = jnp.dot(a, w_ref[...],
                             preferred_element_type=jnp.float32) + b_ref[...]

    out_t = jax.ShapeDtypeStruct((N, Wfc.shape[1]), jnp.float32)
    return pl.pallas_call(body, out_shape=out_t)(aA, aB, Wfc, bfc)


def kernel(x, edge_index, W1, b1, g1, bt1, W2, b2, g2, bt2, Wfc, bfc):
    ei = edge_index.astype(jnp.int32)
    pad = EPAD - E
    pad_src = (jnp.arange(pad, dtype=jnp.int32) * 97) % N
    pad_dst = N + (jnp.arange(pad, dtype=jnp.int32) % 48)
    src2d = jnp.concatenate([ei[0], pad_src]).reshape(ROWS128, 128)
    dst2d = jnp.concatenate([ei[1], pad_dst]).reshape(ROWS128, 128)

    ones128 = jnp.ones((128, HALF), jnp.float32)
    zer128 = jnp.zeros((632, HALF), jnp.float32)

    h1A, h1B = _mm1(x, W1)
    degflat = _deg_parts(dst2d, ones128, zer128)
    z1A, z1B = _scale(h1A, h1B, degflat)
    S1A, S1B = _sc_scatter(z1A, z1B, src2d, dst2d, zer128)
    a1A = _bn_half(S1A, z1A, degflat, b1[:HALF].reshape(1, -1),
                   g1[:HALF].reshape(1, -1), bt1[:HALF].reshape(1, -1))
    a1B = _bn_half(S1B, z1B, degflat, b1[HALF:].reshape(1, -1),
                   g1[HALF:].reshape(1, -1), bt1[HALF:].reshape(1, -1))

    z2A, z2B = _mm_scale2(a1A, a1B, W2, degflat)
    S2A, S2B = _sc_scatter(z2A, z2B, src2d, dst2d, zer128)
    a2A = _bn_half(S2A, z2A, degflat, b2[:HALF].reshape(1, -1),
                   g2[:HALF].reshape(1, -1), bt2[:HALF].reshape(1, -1))
    a2B = _bn_half(S2B, z2B, degflat, b2[HALF:].reshape(1, -1),
                   g2[HALF:].reshape(1, -1), bt2[HALF:].reshape(1, -1))

    return _final(a2A, a2B, Wfc, bfc.reshape(1, -1))


# speedup vs baseline: 16.2315x; 1.0463x over previous
import functools

import jax
import jax.numpy as jnp
from jax import lax
from jax.experimental import pallas as pl
from jax.experimental.pallas import tpu as pltpu
from jax.experimental.pallas import tpu_sc as plsc

N = 10000
E = 320000
EPAD = 327680
ROWS128 = EPAD // 128
CH_SC = ROWS128 // 16
CH_DEG = ROWS128 // 32
ACC = 10112
HALF = 128


def _mesh():
    return plsc.VectorSubcoreMesh(core_axis_name="c", subcore_axis_name="s")


def _deg_parts(dst2d, ones128, zer128):
    @functools.partial(
        pl.kernel,
        out_type=jax.ShapeDtypeStruct((2 * N, HALF), jnp.float32),
        mesh=_mesh(),
        scratch_types=[
            pltpu.VMEM((CH_DEG, 128), jnp.int32),
            pltpu.VMEM((128, HALF), jnp.float32),
            pltpu.VMEM_SHARED((ACC, HALF), jnp.float32),
        ],
    )
    def kern(dst_hbm, ones_hbm, zer_hbm, out_hbm, idx_v, ones_v, acc):
        c = lax.axis_index("c")
        s = lax.axis_index("s")
        wid = s * 2 + c
        pltpu.sync_copy(zer_hbm, acc.at[pl.ds(s * 632, 632)])
        pltpu.sync_copy(ones_hbm, ones_v)
        pltpu.sync_copy(dst_hbm.at[pl.ds(wid * CH_DEG, CH_DEG)], idx_v)
        plsc.subcore_barrier()

        @pl.loop(0, CH_DEG)
        def _(j):
            pltpu.sync_copy(ones_v, acc.at[idx_v.at[j]], add=True)

        plsc.subcore_barrier()

        @pl.when(s < 10)
        def _():
            pltpu.sync_copy(acc.at[pl.ds(s * 1000, 1000)],
                            out_hbm.at[pl.ds(c * N + s * 1000, 1000)])

    return kern(dst2d, ones128, zer128)


def _sc_scatter(zA, zB, src2d, dst2d, zer128):
    out_t = jax.ShapeDtypeStruct((N, HALF), jnp.float32)

    @functools.partial(
        pl.kernel,
        out_type=[out_t, out_t],
        mesh=_mesh(),
        scratch_types=[
            pltpu.VMEM((16, 128), jnp.int32),
            pltpu.VMEM((16, 128), jnp.int32),
            pltpu.VMEM((128, HALF), jnp.float32),
            pltpu.VMEM((128, HALF), jnp.float32),
            pltpu.VMEM_SHARED((ACC, HALF), jnp.float32),
            pltpu.SemaphoreType.DMA,
            pltpu.SemaphoreType.DMA,
            pltpu.SemaphoreType.DMA,
            pltpu.SemaphoreType.DMA,
        ],
    )
    def kern(zA_hbm, zB_hbm, src_hbm, dst_hbm, zer_hbm, outA, outB,
             src_v, dst_v, rows0, rows1, acc, sem0, sem1, sem2, sem3):
        c = lax.axis_index("c")
        s = lax.axis_index("s")
        pltpu.sync_copy(zer_hbm, acc.at[pl.ds(s * 632, 632)])
        plsc.subcore_barrier()

        @pl.loop(0, CH_SC // 2)
        def _(p):
            @pl.when(p > 0)
            def _():
                pltpu.make_async_copy(
                    zA_hbm.at[pl.ds(0, 128)], rows0, sem2).wait()
                pltpu.make_async_copy(
                    zA_hbm.at[pl.ds(0, 128)], rows1, sem3).wait()

            @pl.when(p % 8 == 0)
            def _():
                blk = p // 8
                pltpu.sync_copy(
                    src_hbm.at[pl.ds(s * CH_SC + blk * 16, 16)], src_v)
                pltpu.sync_copy(
                    dst_hbm.at[pl.ds(s * CH_SC + blk * 16, 16)], dst_v)

            r0 = (p % 8) * 2
            r1 = r0 + 1

            @pl.when(c == 0)
            def _():
                cp0 = pltpu.async_copy(zA_hbm.at[src_v.at[r0]], rows0, sem0)
                cp1 = pltpu.async_copy(zA_hbm.at[src_v.at[r1]], rows1, sem1)
                cp0.wait()
                pltpu.async_copy(rows0, acc.at[dst_v.at[r0]], sem2, add=True)
                cp1.wait()
                pltpu.async_copy(rows1, acc.at[dst_v.at[r1]], sem3, add=True)

            @pl.when(c == 1)
            def _():
                cp0 = pltpu.async_copy(zB_hbm.at[src_v.at[r0]], rows0, sem0)
                cp1 = pltpu.async_copy(zB_hbm.at[src_v.at[r1]], rows1, sem1)
                cp0.wait()
                pltpu.async_copy(rows0, acc.at[dst_v.at[r0]], sem2, add=True)
                cp1.wait()
                pltpu.async_copy(rows1, acc.at[dst_v.at[r1]], sem3, add=True)

        pltpu.make_async_copy(zA_hbm.at[pl.ds(0, 128)], rows0, sem2).wait()
        pltpu.make_async_copy(zA_hbm.at[pl.ds(0, 128)], rows1, sem3).wait()
        plsc.subcore_barrier()

        @pl.when((c == 0) & (s < 10))
        def _():
            pltpu.sync_copy(acc.at[pl.ds(s * 1000, 1000)],
                            outA.at[pl.ds(s * 1000, 1000)])

        @pl.when((c == 1) & (s < 10))
        def _():
            pltpu.sync_copy(acc.at[pl.ds(s * 1000, 1000)],
                            outB.at[pl.ds(s * 1000, 1000)])

    return kern(zA, zB, src2d, dst2d, zer128)


def _dinv(deg_ref):
    d = deg_ref[0:N, 0:1] + deg_ref[N:2 * N, 0:1] + 1.0
    return lax.rsqrt(d)


def _mm1(x, W1):
    def body(x_ref, w_ref, hA_ref, hB_ref):
        h = jnp.dot(x_ref[...], w_ref[...], preferred_element_type=jnp.float32)
        hA_ref[...] = h[:, :HALF]
        hB_ref[...] = h[:, HALF:]

    out_t = jax.ShapeDtypeStruct((N, HALF), jnp.float32)
    return pl.pallas_call(body, out_shape=[out_t, out_t])(x, W1)


def _scale(hA, hB, degflat):
    def body(hA_ref, hB_ref, deg_ref, zA_ref, zB_ref, dinv_ref):
        dinv = _dinv(deg_ref)
        zA_ref[...] = hA_ref[...] * dinv
        zB_ref[...] = hB_ref[...] * dinv
        dinv_ref[...] = jnp.broadcast_to(dinv, (N, HALF))

    out_t = jax.ShapeDtypeStruct((N, HALF), jnp.float32)
    return pl.pallas_call(body, out_shape=[out_t, out_t, out_t])(
        hA, hB, degflat)


def _bn(S_ref, z_ref, dinv_ref, b_ref, g_ref, bt_ref):
    p = (S_ref[...] + z_ref[...]) * dinv_ref[...] + b_ref[...]
    mu = jnp.mean(p, axis=0, keepdims=True)
    q = p - mu
    var = jnp.mean(q * q, axis=0, keepdims=True)
    return jnp.maximum(
        q * lax.rsqrt(var + 1e-5) * g_ref[...] + bt_ref[...], 0.0)


def _bn_mm(SA, SB, zA, zB, dinv128, bA, bB, gA, gB, btA, btB, W2):
    def body(SA_ref, SB_ref, zA_ref, zB_ref, dinv_ref, bA_ref, bB_ref,
             gA_ref, gB_ref, btA_ref, btB_ref, w_ref, zoA_ref, zoB_ref):
        aA = _bn(SA_ref, zA_ref, dinv_ref, bA_ref, gA_ref, btA_ref)
        aB = _bn(SB_ref, zB_ref, dinv_ref, bB_ref, gB_ref, btB_ref)
        a = jnp.concatenate([aA, aB], axis=1)
        h = jnp.dot(a, w_ref[...], preferred_element_type=jnp.float32)
        zoA_ref[...] = h[:, :HALF] * dinv_ref[...]
        zoB_ref[...] = h[:, HALF:] * dinv_ref[...]

    out_t = jax.ShapeDtypeStruct((N, HALF), jnp.float32)
    return pl.pallas_call(body, out_shape=[out_t, out_t])(
        SA, SB, zA, zB, dinv128, bA, bB, gA, gB, btA, btB, W2)


def _bn_final(SA, SB, zA, zB, dinv128, bA, bB, gA, gB, btA, btB, Wfc, bfc):
    def body(SA_ref, SB_ref, zA_ref, zB_ref, dinv_ref, bA_ref, bB_ref,
             gA_ref, gB_ref, btA_ref, btB_ref, w_ref, bfc_ref, y_ref):
        aA = _bn(SA_ref, zA_ref, dinv_ref, bA_ref, gA_ref, btA_ref)
        aB = _bn(SB_ref, zB_ref, dinv_ref, bB_ref, gB_ref, btB_ref)
        a = jnp.concatenate([aA, aB], axis=1)
        y_ref[...] = jnp.dot(a, w_ref[...],
                             preferred_element_type=jnp.float32) + bfc_ref[...]

    out_t = jax.ShapeDtypeStruct((N, Wfc.shape[1]), jnp.float32)
    return pl.pallas_call(body, out_shape=out_t)(
        SA, SB, zA, zB, dinv128, bA, bB, gA, gB, btA, btB, Wfc, bfc)


def kernel(x, edge_index, W1, b1, g1, bt1, W2, b2, g2, bt2, Wfc, bfc):
    ei = edge_index.astype(jnp.int32)
    pad = EPAD - E
    pad_src = (jnp.arange(pad, dtype=jnp.int32) * 97) % N
    pad_dst = N + (jnp.arange(pad, dtype=jnp.int32) % 48)
    src2d = jnp.concatenate([ei[0], pad_src]).reshape(ROWS128, 128)
    dst2d = jnp.concatenate([ei[1], pad_dst]).reshape(ROWS128, 128)

    ones128 = jnp.ones((128, HALF), jnp.float32)
    zer128 = jnp.zeros((632, HALF), jnp.float32)

    h1A, h1B = _mm1(x, W1)
    degflat = _deg_parts(dst2d, ones128, zer128)
    z1A, z1B, dinv128 = _scale(h1A, h1B, degflat)
    S1A, S1B = _sc_scatter(z1A, z1B, src2d, dst2d, zer128)
    z2A, z2B = _bn_mm(S1A, S1B, z1A, z1B, dinv128,
                      b1[:HALF].reshape(1, -1), b1[HALF:].reshape(1, -1),
                      g1[:HALF].reshape(1, -1), g1[HALF:].reshape(1, -1),
                      bt1[:HALF].reshape(1, -1), bt1[HALF:].reshape(1, -1),
                      W2)
    S2A, S2B = _sc_scatter(z2A, z2B, src2d, dst2d, zer128)
    return _bn_final(S2A, S2B, z2A, z2B, dinv128,
                     b2[:HALF].reshape(1, -1), b2[HALF:].reshape(1, -1),
                     g2[:HALF].reshape(1, -1), g2[HALF:].reshape(1, -1),
                     bt2[:HALF].reshape(1, -1), bt2[HALF:].reshape(1, -1),
                     Wfc, bfc.reshape(1, -1))
